# baseline (speedup 1.0000x reference)
import functools

import jax
import jax.numpy as jnp
from jax import lax
from jax.experimental import pallas as pl
from jax.experimental.pallas import tpu as pltpu
from jax.experimental.pallas import tpu_sc as plsc

VOCAB = 100000
DEPTH = 64
BATCH = 4
SEQ = 2048
NUM_TOK = BATCH * SEQ
LANES = 16
D_PER_W = DEPTH // 32
XTILE = SEQ // 128


def _emb_body(idx_hbm, tok_hbm, pos_hbm, out_hbm, idx_v, row_v, ob0_v, ob1_v,
              rsem, psem, osem):
    wid = lax.axis_index("s") * 2 + lax.axis_index("c")
    d0 = wid * D_PER_W
    obs = (ob0_v, ob1_v)

    rcopy = pltpu.async_copy(tok_hbm.at[d0], row_v, rsem)
    pltpu.sync_copy(idx_hbm, idx_v)
    pcopies = [
        pltpu.async_copy(pos_hbm.at[d0 + t], obs[t].at[pl.ds(b * SEQ, SEQ)],
                         psem)
        for t in range(D_PER_W) for b in range(BATCH)
    ]

    ocopies = []
    for t in range(D_PER_W):
        ob_v = obs[t]
        rcopy.wait()
        if t == 0:
            for cp in pcopies:
                cp.wait()

        def work(q, _ob=ob_v):
            for b in range(BATCH):
                for k in range(128 // LANES):
                    toks = idx_v[q * BATCH + b, pl.ds(k * LANES, LANES)]
                    vals = plsc.load_gather(row_v, [toks])
                    base = b * SEQ + q * 128 + k * LANES
                    plsc.addupdate(_ob.at[pl.ds(base, LANES)], vals)

        plsc.parallel_loop(0, XTILE, unroll=2)(work)

        if t + 1 < D_PER_W:
            plsc.subcore_barrier()
            rcopy = pltpu.async_copy(tok_hbm.at[d0 + t + 1], row_v, rsem)

        ocopies += [
            pltpu.async_copy(ob_v.at[pl.ds(b * SEQ, SEQ)],
                             out_hbm.at[b * DEPTH + d0 + t], osem)
            for b in range(BATCH)
        ]
    for cp in ocopies:
        cp.wait()


_emb_call = functools.partial(
    pl.kernel,
    mesh=plsc.VectorSubcoreMesh(core_axis_name="c", subcore_axis_name="s"),
    out_type=jax.ShapeDtypeStruct((BATCH * DEPTH, SEQ), jnp.float32),
    scratch_types=[
        pltpu.VMEM((BATCH * XTILE, 128), jnp.int32),
        pltpu.VMEM((VOCAB,), jnp.float32),
        pltpu.VMEM((NUM_TOK,), jnp.float32),
        pltpu.VMEM((NUM_TOK,), jnp.float32),
        pltpu.SemaphoreType.DMA,
        pltpu.SemaphoreType.DMA,
        pltpu.SemaphoreType.DMA,
    ],
    compiler_params=pltpu.CompilerParams(needs_layout_passes=False, skip_device_barrier=True),
)(_emb_body)


def kernel(x, tok_table, pos_table):
    b, s = x.shape
    xq = jnp.transpose(x.reshape(BATCH, XTILE, 128),
                       (1, 0, 2)).reshape(BATCH * XTILE, 128)
    out = _emb_call(xq.astype(jnp.int32), tok_table.T, pos_table.T)
    return out.reshape(b, DEPTH, s).transpose(0, 2, 1)


# speedup vs baseline: 1.0008x; 1.0008x over previous
import functools

import jax
import jax.numpy as jnp
from jax import lax
from jax.experimental import pallas as pl
from jax.experimental.pallas import tpu as pltpu
from jax.experimental.pallas import tpu_sc as plsc

VOCAB = 100000
DEPTH = 64
BATCH = 4
SEQ = 2048
NUM_TOK = BATCH * SEQ
LANES = 16
D_PER_W = DEPTH // 32
XTILE = SEQ // 128


def _emb_body(idx_hbm, tok_hbm, pos_hbm, out_hbm, idx_v, row_v, ob0_v, ob1_v,
              rsem, psem, osem):
    wid = lax.axis_index("s") * 2 + lax.axis_index("c")
    d0 = wid * D_PER_W
    obs = (ob0_v, ob1_v)

    rcopy = pltpu.async_copy(tok_hbm.at[d0], row_v, rsem)
    pltpu.sync_copy(idx_hbm, idx_v)
    pcopies = [
        pltpu.async_copy(pos_hbm.at[d0 + t], obs[t].at[pl.ds(b * SEQ, SEQ)],
                         psem)
        for t in range(D_PER_W) for b in range(BATCH)
    ]

    ocopies = []
    for t in range(D_PER_W):
        ob_v = obs[t]
        rcopy.wait()
        if t == 0:
            for cp in pcopies:
                cp.wait()

        def work(q, _ob=ob_v):
            for b in range(BATCH):
                for k in range(128 // LANES):
                    toks = idx_v[q * BATCH + b, pl.ds(k * LANES, LANES)]
                    vals = plsc.load_gather(row_v, [toks])
                    base = b * SEQ + q * 128 + k * LANES
                    plsc.addupdate(_ob.at[pl.ds(base, LANES)], vals)

        plsc.parallel_loop(0, XTILE, unroll=2)(work)

        plsc.subcore_barrier()

        if t + 1 < D_PER_W:
            rcopy = pltpu.async_copy(tok_hbm.at[d0 + t + 1], row_v, rsem)

        ocopies += [
            pltpu.async_copy(ob_v.at[pl.ds(b * SEQ, SEQ)],
                             out_hbm.at[b * DEPTH + d0 + t], osem)
            for b in range(BATCH)
        ]
    for cp in ocopies:
        cp.wait()


_emb_call = functools.partial(
    pl.kernel,
    mesh=plsc.VectorSubcoreMesh(core_axis_name="c", subcore_axis_name="s"),
    out_type=jax.ShapeDtypeStruct((BATCH * DEPTH, SEQ), jnp.float32),
    scratch_types=[
        pltpu.VMEM((BATCH * XTILE, 128), jnp.int32),
        pltpu.VMEM((VOCAB,), jnp.float32),
        pltpu.VMEM((NUM_TOK,), jnp.float32),
        pltpu.VMEM((NUM_TOK,), jnp.float32),
        pltpu.SemaphoreType.DMA,
        pltpu.SemaphoreType.DMA,
        pltpu.SemaphoreType.DMA,
    ],
    compiler_params=pltpu.CompilerParams(needs_layout_passes=False),
)(_emb_body)


def kernel(x, tok_table, pos_table):
    b, s = x.shape
    xq = jnp.transpose(x.reshape(BATCH, XTILE, 128),
                       (1, 0, 2)).reshape(BATCH * XTILE, 128)
    out = _emb_call(xq.astype(jnp.int32), tok_table.T, pos_table.T)
    return out.reshape(b, DEPTH, s).transpose(0, 2, 1)


# speedup vs baseline: 1.0459x; 1.0451x over previous
import functools

import jax
import jax.numpy as jnp
from jax import lax
from jax.experimental import pallas as pl
from jax.experimental.pallas import tpu as pltpu
from jax.experimental.pallas import tpu_sc as plsc

VOCAB = 100000
DEPTH = 64
BATCH = 4
SEQ = 2048
NUM_TOK = BATCH * SEQ
LANES = 16
D_PER_W = DEPTH // 32
XTILE = SEQ // 128


def _emb_body(idx_hbm, tok_hbm, pos_hbm, out_hbm, idx_v, row_v, ob0_v, ob1_v,
              rsem, psem, osem):
    wid = lax.axis_index("s") * 2 + lax.axis_index("c")
    d0 = wid * D_PER_W
    obs = (ob0_v, ob1_v)

    rcopy = pltpu.async_copy(tok_hbm.at[d0], row_v, rsem)
    pltpu.sync_copy(idx_hbm, idx_v)
    pcopies = [
        pltpu.async_copy(pos_hbm.at[d0 + t], obs[t].at[b], psem)
        for t in range(D_PER_W) for b in range(BATCH)
    ]

    ocopies = []
    for t in range(D_PER_W):
        ob_v = obs[t]
        rcopy.wait()
        if t == 0:
            for cp in pcopies:
                cp.wait()

        def work(q, _ob=ob_v):
            for b in range(BATCH):
                for k in range(128 // LANES):
                    toks = idx_v[q * BATCH + b, pl.ds(k * LANES, LANES)]
                    vals = plsc.load_gather(row_v, [toks])
                    sl = pl.ds(q * 128 + k * LANES, LANES)
                    plsc.addupdate(_ob.at[b, sl], vals)

        plsc.parallel_loop(0, XTILE, unroll=2)(work)

        plsc.subcore_barrier()

        if t + 1 < D_PER_W:
            rcopy = pltpu.async_copy(tok_hbm.at[d0 + t + 1], row_v, rsem)

        ocopies.append(
            pltpu.async_copy(ob_v, out_hbm.at[:, d0 + t], osem))
    for cp in ocopies:
        cp.wait()


_emb_call = functools.partial(
    pl.kernel,
    mesh=plsc.VectorSubcoreMesh(core_axis_name="c", subcore_axis_name="s"),
    out_type=jax.ShapeDtypeStruct((BATCH, DEPTH, SEQ), jnp.float32),
    scratch_types=[
        pltpu.VMEM((BATCH * XTILE, 128), jnp.int32),
        pltpu.VMEM((VOCAB,), jnp.float32),
        pltpu.VMEM((BATCH, SEQ), jnp.float32),
        pltpu.VMEM((BATCH, SEQ), jnp.float32),
        pltpu.SemaphoreType.DMA,
        pltpu.SemaphoreType.DMA,
        pltpu.SemaphoreType.DMA,
    ],
    compiler_params=pltpu.CompilerParams(needs_layout_passes=False),
)(_emb_body)


def kernel(x, tok_table, pos_table):
    b, s = x.shape
    xq = jnp.transpose(x.reshape(BATCH, XTILE, 128),
                       (1, 0, 2)).reshape(BATCH * XTILE, 128)
    out = _emb_call(xq.astype(jnp.int32), tok_table.T, pos_table.T)
    return out.transpose(0, 2, 1)
